# baseline (speedup 1.0000x reference)
import functools

import jax
import jax.numpy as jnp
from jax import lax
from jax.experimental import pallas as pl
from jax.experimental.pallas import tpu as pltpu, tpu_sc as plsc

_info = plsc.get_sparse_core_info()
_NC, _NS, _L = _info.num_cores, _info.num_subcores, _info.num_lanes
_NW = _NC * _NS


@functools.partial(jax.jit, static_argnums=(2, 3, 4, 5))
def _warp_sc(xf, uf, B, T, M, N):
    ROWS = B * T * M
    RPW = ROWS // _NW
    CH = 16
    NCHUNK = RPW // CH
    VPC = CH * N // _L
    VPR = N // _L
    PIX = M * N

    assert ROWS % _NW == 0 and RPW % CH == 0 and (T * M) % RPW == 0

    mesh = plsc.VectorSubcoreMesh(core_axis_name="c", subcore_axis_name="s")

    @functools.partial(
        pl.kernel,
        out_type=jax.ShapeDtypeStruct((ROWS * N,), jnp.float32),
        mesh=mesh,
        scratch_types=[
            pltpu.VMEM((PIX,), jnp.float32),
            pltpu.VMEM((CH * N * 2,), jnp.float32),
            pltpu.VMEM((CH * N,), jnp.float32),
        ],
        compiler_params=pltpu.CompilerParams(needs_layout_passes=False),
    )
    def body(x_hbm, u_hbm, out_hbm, img, flow, outb):
        wid = lax.axis_index("s") * _NC + lax.axis_index("c")
        s0 = wid * RPW
        bidx = s0 // (T * M)
        pltpu.sync_copy(x_hbm.at[pl.ds(bidx * PIX, PIX)], img)

        lane = lax.iota(jnp.int32, _L)

        def chunk_body(k, _):
            srow = s0 + k * CH
            pltpu.sync_copy(u_hbm.at[pl.ds(srow * N * 2, CH * N * 2)], flow)
            i0 = srow % M

            def vec_body(v, _):
                roff = v >> 4
                jbase = (v - (roff << 4)) * _L
                pix0 = v * _L
                pvec = jnp.full((_L,), pix0 * 2, jnp.int32) + lane * 2
                ux = plsc.load_gather(flow, [pvec])
                uy = plsc.load_gather(flow, [pvec + 1])

                jf = (jnp.full((_L,), jbase, jnp.int32) + lane).astype(jnp.float32)
                i_f = jnp.full((_L,), (i0 + roff).astype(jnp.float32))

                fx = jnp.minimum(jnp.maximum(jf + ux, -4.0), 300.0)
                fy = jnp.minimum(jnp.maximum(i_f + uy, -4.0), 300.0)

                xt = fx.astype(jnp.int32)
                yt = fy.astype(jnp.int32)
                x0 = jnp.where(xt.astype(jnp.float32) > fx, xt - 1, xt)
                y0 = jnp.where(yt.astype(jnp.float32) > fy, yt - 1, yt)
                wx = fx - x0.astype(jnp.float32)
                wy = fy - y0.astype(jnp.float32)

                x1 = x0 + 1
                y1 = y0 + 1
                mx0 = x0.astype(jnp.uint32) < jnp.uint32(N)
                mx1 = x1.astype(jnp.uint32) < jnp.uint32(N)
                my0 = y0.astype(jnp.uint32) < jnp.uint32(M)
                my1 = y1.astype(jnp.uint32) < jnp.uint32(M)

                lin00 = ((y0 << 8) + x0) & (PIX - 1)
                lin01 = (lin00 + 1) & (PIX - 1)
                lin10 = (lin00 + N) & (PIX - 1)
                lin11 = (lin01 + N) & (PIX - 1)

                v00 = jnp.where(mx0 & my0, plsc.load_gather(img, [lin00]), 0.0)
                v01 = jnp.where(mx1 & my0, plsc.load_gather(img, [lin01]), 0.0)
                v10 = jnp.where(mx0 & my1, plsc.load_gather(img, [lin10]), 0.0)
                v11 = jnp.where(mx1 & my1, plsc.load_gather(img, [lin11]), 0.0)

                owx = 1.0 - wx
                owy = 1.0 - wy
                res = (v00 * owx * owy + v01 * wx * owy
                       + v10 * owx * wy + v11 * wx * wy)
                outb[pl.ds(pix0, _L)] = res
                return 0

            lax.fori_loop(0, VPC, vec_body, 0)
            pltpu.sync_copy(outb, out_hbm.at[pl.ds(srow * N, CH * N)])
            return 0

        lax.fori_loop(0, NCHUNK, chunk_body, 0)

    return body(xf, uf)


def kernel(x, u):
    B, T, M, N, _ = u.shape
    out = _warp_sc(x.reshape(-1), u.reshape(-1), B, T, M, N)
    return out.reshape(B, T, M, N)


# speedup vs baseline: 26.0203x; 26.0203x over previous
import functools

import jax
import jax.numpy as jnp
from jax import lax
from jax.experimental import pallas as pl
from jax.experimental.pallas import tpu as pltpu, tpu_sc as plsc

_info = plsc.get_sparse_core_info()
_NC, _NS, _L = _info.num_cores, _info.num_subcores, _info.num_lanes
_NW = _NC * _NS

_PADY = 2
_PADX = 8


@functools.partial(jax.jit, static_argnums=(2, 3, 4, 5))
def _warp_sc(x1d, u1d, B, T, M, N):
    ROWS = B * T * M
    RPW = ROWS // _NW
    CH = 16
    NCHUNK = RPW // CH
    VPC = CH * N // _L
    STRIDE = N + _PADX
    PROWS = M + _PADY + 4
    PPIX = PROWS * STRIDE
    PIX = M * N
    QTR = PIX // 4

    assert ROWS % _NW == 0 and RPW % CH == 0 and (T * M) % RPW == 0
    assert NCHUNK % 2 == 0 and N == 256 and M == 256

    mesh = plsc.VectorSubcoreMesh(core_axis_name="c", subcore_axis_name="s")

    XLO, XHI = float(_PADX - 2), float(_PADX + N + 1)
    YLO, YHI = float(_PADY - 2), float(_PADY + M + 1)

    @functools.partial(
        pl.kernel,
        out_type=jax.ShapeDtypeStruct((ROWS * N,), jnp.float32),
        mesh=mesh,
        scratch_types=[
            pltpu.VMEM((PPIX,), jnp.float32),
            pltpu.VMEM((QTR,), jnp.float32),
            pltpu.VMEM((CH * N * 2,), jnp.float32),
            pltpu.VMEM((CH * N * 2,), jnp.float32),
            pltpu.VMEM((CH * N,), jnp.float32),
            pltpu.VMEM((CH * N,), jnp.float32),
            pltpu.SemaphoreType.DMA,
            pltpu.SemaphoreType.DMA,
            pltpu.SemaphoreType.DMA,
            pltpu.SemaphoreType.DMA,
        ],
        compiler_params=pltpu.CompilerParams(needs_layout_passes=False),
    )
    def body(x_hbm, u_hbm, out_hbm, img, raw, fl0, fl1, ob0, ob1,
             si0, si1, so0, so1):
        wid = lax.axis_index("s") * _NC + lax.axis_index("c")
        s0 = wid * RPW
        bidx = s0 // (T * M)

        zero16 = jnp.zeros((_L,), jnp.float32)
        lane = lax.iota(jnp.int32, _L)
        lanef = lane.astype(jnp.float32)

        def zero_body(t, _):
            img[pl.ds(t * _L, _L)] = zero16
            return 0
        lax.fori_loop(0, PPIX // _L, zero_body, 0)

        def qtr_body(q, _):
            pltpu.sync_copy(x_hbm.at[pl.ds(bidx * PIX + q * QTR, QTR)], raw)

            def detile_body(t, _):
                iblk = q * 8 + (t >> 7)
                r = t & 127
                jblk = r >> 6
                i8 = (r >> 3) & 7
                k = r & 7
                dst = (iblk * 8 + i8 + _PADY) * STRIDE + _PADX \
                    + jblk * 128 + k * _L
                img[pl.ds(dst, _L)] = raw[pl.ds(t * _L, _L)]
                return 0
            lax.fori_loop(0, QTR // _L, detile_body, 0)
            return 0
        lax.fori_loop(0, 4, qtr_body, 0)

        flbufs = (fl0, fl1)
        obufs = (ob0, ob1)
        isems = (si0, si1)
        osems = (so0, so1)

        def start_in(c, buf, sem):
            pltpu.async_copy(
                u_hbm.at[pl.ds((s0 + c * CH) * N * 2, CH * N * 2)], buf, sem)

        start_in(0, fl0, si0)

        def pair_body(kp, _):
            k0 = kp * 2
            for ph in range(2):
                k = k0 + ph
                flow, outb = flbufs[ph], obufs[ph]
                isem, osem = isems[ph], osems[ph]
                srow = s0 + k * CH
                nxt = k + 1

                @pl.when(nxt < NCHUNK)
                def _():
                    start_in(nxt, flbufs[1 - ph], isems[1 - ph])

                pltpu.make_async_copy(
                    u_hbm.at[pl.ds(srow * N * 2, CH * N * 2)], flow, isem
                ).wait()

                @pl.when(k >= 2)
                def _():
                    pltpu.make_async_copy(
                        outb, out_hbm.at[pl.ds(srow * N, CH * N)], osem
                    ).wait()

                i0 = srow % M

                def vec_body(v, _):
                    roff = v >> 4
                    vj = v & 15
                    fo = roff * 512 + (vj >> 3) * 256 + (vj & 7) * _L
                    ux = flow[pl.ds(fo, _L)]
                    uy = flow[pl.ds(fo + 128, _L)]

                    jf = jnp.full(
                        (_L,), (vj * _L + _PADX).astype(jnp.float32)) + lanef
                    i_f = jnp.full(
                        (_L,), (i0 + roff + _PADY).astype(jnp.float32))

                    fx = jnp.minimum(jnp.maximum(jf + ux, XLO), XHI)
                    fy = jnp.minimum(jnp.maximum(i_f + uy, YLO), YHI)

                    xc = fx.astype(jnp.int32)
                    yc = fy.astype(jnp.int32)
                    wx = fx - xc.astype(jnp.float32)
                    wy = fy - yc.astype(jnp.float32)

                    lin00 = (yc << 8) + (yc << 3) + xc
                    lin01 = lin00 + 1
                    lin10 = lin00 + STRIDE
                    lin11 = lin10 + 1

                    v00 = plsc.load_gather(img, [lin00])
                    v01 = plsc.load_gather(img, [lin01])
                    v10 = plsc.load_gather(img, [lin10])
                    v11 = plsc.load_gather(img, [lin11])

                    owx = 1.0 - wx
                    owy = 1.0 - wy
                    res = (v00 * owx * owy + v01 * wx * owy
                           + v10 * owx * wy + v11 * wx * wy)
                    opos = (roff >> 3) * 2048 + (vj >> 3) * 1024 \
                        + (roff & 7) * 128 + (vj & 7) * _L
                    outb[pl.ds(opos, _L)] = res
                    return 0

                lax.fori_loop(0, VPC, vec_body, 0)
                pltpu.async_copy(outb, out_hbm.at[pl.ds(srow * N, CH * N)],
                                 osem)
            return 0

        lax.fori_loop(0, NCHUNK // 2, pair_body, 0)

        for ph in range(2):
            srow_last = s0 + (NCHUNK - 2 + ph) * CH
            pltpu.make_async_copy(
                obufs[ph], out_hbm.at[pl.ds(srow_last * N, CH * N)], osems[ph]
            ).wait()

    return body(x1d, u1d)


def kernel(x, u):
    B, T, M, N, _ = u.shape
    x1d = (x.reshape(B, M // 8, 8, N // 128, 128)
            .transpose(0, 1, 3, 2, 4).reshape(-1))
    u1d = (u.reshape(B, T, M, N // 128, 128, 2)
            .transpose(0, 1, 2, 3, 5, 4).reshape(-1))
    out = _warp_sc(x1d, u1d, B, T, M, N)
    return (out.reshape(B, T, M // 8, N // 128, 8, 128)
              .transpose(0, 1, 2, 4, 3, 5).reshape(B, T, M, N))


# speedup vs baseline: 60.0539x; 2.3080x over previous
import functools

import jax
import jax.numpy as jnp
from jax import lax
from jax.experimental import pallas as pl
from jax.experimental.pallas import tpu as pltpu, tpu_sc as plsc

_info = plsc.get_sparse_core_info()
_NC, _NS, _L = _info.num_cores, _info.num_subcores, _info.num_lanes
_NW = _NC * _NS

_PADY = 2
_PADX = 8


@functools.partial(jax.jit, static_argnums=(2, 3, 4, 5))
def _warp_sc(x1d, u1d, B, T, M, N):
    ROWS = B * T * M
    RPW = ROWS // _NW
    CH = 16
    NCHUNK = RPW // CH
    VPC = CH * N // _L
    STRIDE = N + _PADX
    PROWS = M + _PADY + 4
    PPIX = -(-(PROWS * STRIDE) // (4 * _L)) * (4 * _L)
    PIX = M * N
    QTR = PIX // 4

    assert ROWS % _NW == 0 and RPW % CH == 0 and (T * M) % RPW == 0
    assert NCHUNK % 2 == 0 and N == 256 and M == 256

    mesh = plsc.VectorSubcoreMesh(core_axis_name="c", subcore_axis_name="s")

    XLO, XHI = float(_PADX - 2), float(_PADX + N + 1)
    YLO, YHI = float(_PADY - 2), float(_PADY + M + 1)

    @functools.partial(
        pl.kernel,
        out_type=jax.ShapeDtypeStruct((ROWS * N,), jnp.float32),
        mesh=mesh,
        scratch_types=[
            pltpu.VMEM((PPIX,), jnp.float32),
            pltpu.VMEM((QTR,), jnp.float32),
            pltpu.VMEM((CH * N * 2,), jnp.float32),
            pltpu.VMEM((CH * N * 2,), jnp.float32),
            pltpu.VMEM((CH * N,), jnp.float32),
            pltpu.VMEM((CH * N,), jnp.float32),
            pltpu.SemaphoreType.DMA,
            pltpu.SemaphoreType.DMA,
            pltpu.SemaphoreType.DMA,
            pltpu.SemaphoreType.DMA,
        ],
        compiler_params=pltpu.CompilerParams(needs_layout_passes=False),
    )
    def body(x_hbm, u_hbm, out_hbm, img, raw, fl0, fl1, ob0, ob1,
             si0, si1, so0, so1):
        wid = lax.axis_index("s") * _NC + lax.axis_index("c")
        s0 = wid * RPW
        bidx = s0 // (T * M)

        zero16 = jnp.zeros((_L,), jnp.float32)
        lane = lax.iota(jnp.int32, _L)
        lanef = lane.astype(jnp.float32)
        lanefs = [lanef + float(_PADX + s * _L) for s in range(4)]

        def zero_body(t, _):
            for s in range(4):
                img[pl.ds(t * 4 * _L + s * _L, _L)] = zero16
            return 0
        lax.fori_loop(0, PPIX // (4 * _L), zero_body, 0)

        def qtr_body(q, _):
            pltpu.sync_copy(x_hbm.at[pl.ds(bidx * PIX + q * QTR, QTR)], raw)

            def detile_body(g, _):
                t = g * 4
                iblk = q * 8 + (t >> 7)
                r = t & 127
                jblk = r >> 6
                i8 = (r >> 3) & 7
                k = r & 7
                dst = (iblk * 8 + i8 + _PADY) * STRIDE + _PADX \
                    + jblk * 128 + k * _L
                for s in range(4):
                    img[pl.ds(dst + s * _L, _L)] = raw[pl.ds(t * _L + s * _L, _L)]
                return 0
            lax.fori_loop(0, QTR // (4 * _L), detile_body, 0)
            return 0
        lax.fori_loop(0, 4, qtr_body, 0)

        flbufs = (fl0, fl1)
        obufs = (ob0, ob1)
        isems = (si0, si1)
        osems = (so0, so1)

        def start_in(c, buf, sem):
            pltpu.async_copy(
                u_hbm.at[pl.ds((s0 + c * CH) * N * 2, CH * N * 2)], buf, sem)

        start_in(0, fl0, si0)

        def pair_body(kp, _):
            k0 = kp * 2
            for ph in range(2):
                k = k0 + ph
                flow, outb = flbufs[ph], obufs[ph]
                isem, osem = isems[ph], osems[ph]
                srow = s0 + k * CH
                nxt = k + 1

                @pl.when(nxt < NCHUNK)
                def _():
                    start_in(nxt, flbufs[1 - ph], isems[1 - ph])

                pltpu.make_async_copy(
                    u_hbm.at[pl.ds(srow * N * 2, CH * N * 2)], flow, isem
                ).wait()

                @pl.when(k >= 2)
                def _():
                    pltpu.make_async_copy(
                        outb, out_hbm.at[pl.ds(srow * N, CH * N)], osem
                    ).wait()

                i0 = srow % M

                def vec_body(g, _):
                    v = g * 4
                    roff = v >> 4
                    vj0 = v & 15
                    fo = roff * 512 + (vj0 >> 3) * 256 + (vj0 & 7) * _L
                    opos = (roff >> 3) * 2048 + (vj0 >> 3) * 1024 \
                        + (roff & 7) * 128 + (vj0 & 7) * _L
                    jb = jnp.full((_L,), (vj0 * _L).astype(jnp.float32))
                    i_f = jnp.full(
                        (_L,), (i0 + roff + _PADY).astype(jnp.float32))

                    R = range(4)
                    ux = [flow[pl.ds(fo + s * _L, _L)] for s in R]
                    uy = [flow[pl.ds(fo + s * _L + 128, _L)] for s in R]
                    fx = [jnp.minimum(
                        jnp.maximum((ux[s] + lanefs[s]) + jb, XLO), XHI)
                        for s in R]
                    fy = [jnp.minimum(jnp.maximum(i_f + uy[s], YLO), YHI)
                          for s in R]
                    xc = [fx[s].astype(jnp.int32) for s in R]
                    yc = [fy[s].astype(jnp.int32) for s in R]
                    wx = [fx[s] - xc[s].astype(jnp.float32) for s in R]
                    wy = [fy[s] - yc[s].astype(jnp.float32) for s in R]
                    l00 = [yc[s] * STRIDE + xc[s] for s in R]
                    l10 = [l00[s] + STRIDE for s in R]
                    v00 = [plsc.load_gather(img, [l00[s]]) for s in R]
                    v01 = [plsc.load_gather(img, [l00[s] + 1]) for s in R]
                    v10 = [plsc.load_gather(img, [l10[s]]) for s in R]
                    v11 = [plsc.load_gather(img, [l10[s] + 1]) for s in R]
                    for s in R:
                        top = v00[s] + wx[s] * (v01[s] - v00[s])
                        bot = v10[s] + wx[s] * (v11[s] - v10[s])
                        res = top + wy[s] * (bot - top)
                        outb[pl.ds(opos + s * _L, _L)] = res
                    return 0

                lax.fori_loop(0, VPC // 4, vec_body, 0)
                pltpu.async_copy(outb, out_hbm.at[pl.ds(srow * N, CH * N)],
                                 osem)
            return 0

        lax.fori_loop(0, NCHUNK // 2, pair_body, 0)

        for ph in range(2):
            srow_last = s0 + (NCHUNK - 2 + ph) * CH
            pltpu.make_async_copy(
                obufs[ph], out_hbm.at[pl.ds(srow_last * N, CH * N)], osems[ph]
            ).wait()

    return body(x1d, u1d)


def kernel(x, u):
    B, T, M, N, _ = u.shape
    x1d = (x.reshape(B, M // 8, 8, N // 128, 128)
            .transpose(0, 1, 3, 2, 4).reshape(-1))
    u1d = (u.reshape(B, T, M, N // 128, 128, 2)
            .transpose(0, 1, 2, 3, 5, 4).reshape(-1))
    out = _warp_sc(x1d, u1d, B, T, M, N)
    return (out.reshape(B, T, M // 8, N // 128, 8, 128)
              .transpose(0, 1, 2, 4, 3, 5).reshape(B, T, M, N))
